# baseline (speedup 1.0000x reference)
import functools

import jax
import jax.numpy as jnp
from jax import lax
from jax.experimental import pallas as pl
from jax.experimental.pallas import tpu as pltpu
from jax.experimental.pallas import tpu_sc as plsc

NUM_CORES = 2
NUM_SUBCORES = 16
NUM_WORKERS = NUM_CORES * NUM_SUBCORES
LANES = 16


@jax.jit
def _bow_encode(input_ids, emb_weight):
    B, S = input_ids.shape
    V, E = emb_weight.shape
    b_per_w = B // NUM_WORKERS
    n_chunks = 5
    chunk = S // n_chunks
    n_col = E // LANES

    mesh = plsc.VectorSubcoreMesh(
        core_axis_name="c", subcore_axis_name="s",
        num_cores=NUM_CORES, num_subcores=NUM_SUBCORES,
    )

    @functools.partial(
        pl.kernel,
        out_type=jax.ShapeDtypeStruct((B, E), jnp.float32),
        mesh=mesh,
        scratch_types=[
            pltpu.VMEM((b_per_w, S), jnp.int32),
            pltpu.VMEM((S, E), jnp.float32),
            pltpu.VMEM((S, E), jnp.float32),
            pltpu.VMEM((b_per_w, E), jnp.float32),
            pltpu.SemaphoreType.DMA,
            pltpu.SemaphoreType.DMA,
        ],
        compiler_params=pltpu.CompilerParams(use_tc_tiling_on_sc=False),
    )
    def bow_kernel(idx_hbm, table_hbm, out_hbm, idx_v, buf0, buf1, out_v,
                   sem0, sem1):
        wid = lax.axis_index("s") * NUM_CORES + lax.axis_index("c")
        base = wid * b_per_w
        pltpu.sync_copy(idx_hbm.at[pl.ds(base, b_per_w), :], idx_v)

        bufs = (buf0, buf1)
        sems = (sem0, sem1)

        def fire(b, p):
            for j in range(n_chunks):
                pltpu.async_copy(
                    table_hbm.at[idx_v.at[b, pl.ds(j * chunk, chunk)]],
                    bufs[p].at[pl.ds(j * chunk, chunk), :],
                    sems[p],
                )

        def drain(p):
            pltpu.make_async_copy(
                table_hbm.at[pl.ds(0, S), :], bufs[p], sems[p]).wait()

        def reduce(p, b):
            buf = bufs[p]

            def body(s, accs):
                return tuple(
                    jnp.maximum(a, buf[s, pl.ds(j * LANES, LANES)])
                    for j, a in enumerate(accs)
                )

            accs = tuple(buf[0, pl.ds(j * LANES, LANES)] for j in range(n_col))
            accs = lax.fori_loop(1, S, body, accs, unroll=4)
            for j in range(n_col):
                out_v[b, pl.ds(j * LANES, LANES)] = accs[j]

        fire(0, 0)

        def pair_body(i):
            b0 = 2 * i
            fire(b0 + 1, 1)
            drain(0)
            reduce(0, b0)
            fire(b0 + 2, 0)
            drain(1)
            reduce(1, b0 + 1)

        pl.loop(0, b_per_w // 2 - 1)(pair_body)

        b0 = b_per_w - 2
        fire(b0 + 1, 1)
        drain(0)
        reduce(0, b0)
        drain(1)
        reduce(1, b0 + 1)

        pltpu.sync_copy(out_v, out_hbm.at[pl.ds(base, b_per_w), :])

    return bow_kernel(input_ids, emb_weight)


_BLK = 4096


@jax.jit
def _relayout(embT):
    E, V = embT.shape
    nfull = V // _BLK
    tail = (V - nfull * _BLK) // 128 * 128
    rag = V - nfull * _BLK - tail

    def body(x_hbm, patch_hbm, out_hbm, x0, x1, y0, y1, si0, si1, so0, so1):
        xs, ys = (x0, x1), (y0, y1)
        sis, sos = (si0, si1), (so0, so1)

        rows = lax.broadcasted_iota(jnp.int32, (E, E), 0)
        cols = lax.broadcasted_iota(jnp.int32, (E, E), 1)
        eye = (rows == cols).astype(jnp.float32)

        def fire_in(b, p):
            pltpu.async_copy(
                x_hbm.at[:, pl.ds(b * _BLK, _BLK)], xs[p], sis[p])

        def wait_in(p):
            pltpu.make_async_copy(
                x_hbm.at[:, pl.ds(0, _BLK)], xs[p], sis[p]).wait()

        def step(b, p, first):
            wait_in(p)
            y = jax.lax.dot_general(
                xs[p][...], eye, (((0,), (0,)), ((), ())),
                precision=jax.lax.Precision.HIGHEST)
            @pl.when(jnp.logical_not(first))
            def _():
                pltpu.make_async_copy(
                    ys[p], out_hbm.at[pl.ds(0, _BLK), :], sos[p]).wait()
            ys[p][:, :E] = y
            ys[p][:, E:] = jnp.zeros((_BLK, E), jnp.float32)
            pltpu.async_copy(
                ys[p], out_hbm.at[pl.ds(b * _BLK, _BLK), :], sos[p])

        fire_in(0, 0)
        fire_in(1, 1)

        def pair(i):
            b0 = 2 * i
            step(b0, 0, i == 0)
            @pl.when(b0 + 2 < nfull)
            def _():
                fire_in(b0 + 2, 0)
            step(b0 + 1, 1, i == 0)
            @pl.when(b0 + 3 < nfull)
            def _():
                fire_in(b0 + 3, 1)

        pl.loop(0, nfull // 2)(pair)

        base = nfull * _BLK
        pltpu.async_copy(
            x_hbm.at[:, pl.ds(base, tail)], xs[1].at[:, pl.ds(0, tail)], si1)
        pltpu.make_async_copy(
            x_hbm.at[:, pl.ds(0, tail)], xs[1].at[:, pl.ds(0, tail)],
            si1).wait()
        yt = jax.lax.dot_general(
            xs[1][:, pl.ds(0, tail)], eye, (((0,), (0,)), ((), ())),
            precision=jax.lax.Precision.HIGHEST)
        pltpu.make_async_copy(
            ys[1], out_hbm.at[pl.ds(0, _BLK), :], so1).wait()
        ys[1][:tail, :E] = yt
        ys[1][:tail, E:] = jnp.zeros((tail, E), jnp.float32)
        pltpu.async_copy(
            ys[1].at[pl.ds(0, tail), :], out_hbm.at[pl.ds(base, tail), :],
            so1)
        pltpu.sync_copy(patch_hbm, out_hbm.at[pl.ds(base + tail, rag), :])
        pltpu.make_async_copy(
            ys[0], out_hbm.at[pl.ds(0, _BLK), :], so0).wait()
        pltpu.make_async_copy(
            ys[1].at[pl.ds(0, tail), :], out_hbm.at[pl.ds(0, tail), :],
            so1).wait()

    patch = jnp.pad(
        jax.lax.dot_general(
            embT[:, V - rag:], jnp.eye(E, dtype=jnp.float32),
            (((0,), (0,)), ((), ())),
            precision=jax.lax.Precision.HIGHEST),
        ((0, 0), (0, E)))
    out = pl.pallas_call(
        body,
        in_specs=[pl.BlockSpec(memory_space=pl.ANY),
                  pl.BlockSpec(memory_space=pl.ANY)],
        out_specs=pl.BlockSpec(memory_space=pl.ANY),
        out_shape=jax.ShapeDtypeStruct((V, 2 * E), jnp.float32),
        scratch_shapes=[
            pltpu.VMEM((E, _BLK), jnp.float32),
            pltpu.VMEM((E, _BLK), jnp.float32),
            pltpu.VMEM((_BLK, 2 * E), jnp.float32),
            pltpu.VMEM((_BLK, 2 * E), jnp.float32),
            pltpu.SemaphoreType.DMA,
            pltpu.SemaphoreType.DMA,
            pltpu.SemaphoreType.DMA,
            pltpu.SemaphoreType.DMA,
        ],
    )(embT, patch)
    return out.reshape(2 * V, E)


def kernel(input, emb_weight):
    emb2 = _relayout(emb_weight.T)
    return _bow_encode(input.astype(jnp.int32) * 2, emb2)


# speedup vs baseline: 2.3281x; 2.3281x over previous
import functools

import jax
import jax.numpy as jnp
from jax import lax
from jax.experimental import pallas as pl
from jax.experimental.pallas import tpu as pltpu
from jax.experimental.pallas import tpu_sc as plsc

NUM_CORES = 2
NUM_SUBCORES = 16
NUM_WORKERS = NUM_CORES * NUM_SUBCORES
LANES = 16


@jax.jit
def _bow_encode(input_ids, emb_weight):
    B, S = input_ids.shape
    V, E = emb_weight.shape
    b_per_w = B // NUM_WORKERS
    n_chunks = 5
    chunk = S // n_chunks
    n_col = E // LANES

    mesh = plsc.VectorSubcoreMesh(
        core_axis_name="c", subcore_axis_name="s",
        num_cores=NUM_CORES, num_subcores=NUM_SUBCORES,
    )

    @functools.partial(
        pl.kernel,
        out_type=jax.ShapeDtypeStruct((B, E), jnp.float32),
        mesh=mesh,
        scratch_types=[
            pltpu.VMEM((b_per_w, S), jnp.int32),
            pltpu.VMEM((S, E), jnp.float32),
            pltpu.VMEM((S, E), jnp.float32),
            pltpu.VMEM((b_per_w, E), jnp.float32),
            pltpu.SemaphoreType.DMA,
            pltpu.SemaphoreType.DMA,
        ],
        compiler_params=pltpu.CompilerParams(use_tc_tiling_on_sc=False),
    )
    def bow_kernel(idx_hbm, table_hbm, out_hbm, idx_v, buf0, buf1, out_v,
                   sem0, sem1):
        wid = lax.axis_index("s") * NUM_CORES + lax.axis_index("c")
        base = wid * b_per_w
        pltpu.sync_copy(idx_hbm.at[pl.ds(base, b_per_w), :], idx_v)

        bufs = (buf0, buf1)
        sems = (sem0, sem1)

        def fire(b, p):
            for j in range(n_chunks):
                pltpu.async_copy(
                    table_hbm.at[idx_v.at[b, pl.ds(j * chunk, chunk)]],
                    bufs[p].at[pl.ds(j * chunk, chunk), :],
                    sems[p],
                )

        def drain(p):
            pltpu.make_async_copy(
                table_hbm.at[pl.ds(0, S), :], bufs[p], sems[p]).wait()

        def reduce(p, b):
            buf = bufs[p]

            def body(s, accs):
                return tuple(
                    jnp.maximum(a, buf[s, pl.ds(j * LANES, LANES)])
                    for j, a in enumerate(accs)
                )

            accs = tuple(buf[0, pl.ds(j * LANES, LANES)] for j in range(n_col))
            accs = lax.fori_loop(1, S, body, accs, unroll=4)
            for j in range(n_col):
                out_v[b, pl.ds(j * LANES, LANES)] = accs[j]

        fire(0, 0)

        def pair_body(i):
            b0 = 2 * i
            fire(b0 + 1, 1)
            drain(0)
            reduce(0, b0)
            fire(b0 + 2, 0)
            drain(1)
            reduce(1, b0 + 1)

        pl.loop(0, b_per_w // 2 - 1)(pair_body)

        b0 = b_per_w - 2
        fire(b0 + 1, 1)
        drain(0)
        reduce(0, b0)
        drain(1)
        reduce(1, b0 + 1)

        pltpu.sync_copy(out_v, out_hbm.at[pl.ds(base, b_per_w), :])

    return bow_kernel(input_ids, emb_weight)


_BLK = 4096


@jax.jit
def _relayout(embT):
    E, V = embT.shape
    nfull = V // _BLK
    tail = (V - nfull * _BLK) // 128 * 128
    rag = V - nfull * _BLK - tail

    def body(x_hbm, patch_hbm, out_hbm, x0, x1, y0, y1, si0, si1, so0, so1):
        xs, ys = (x0, x1), (y0, y1)
        sis, sos = (si0, si1), (so0, so1)

        rows = lax.broadcasted_iota(jnp.int32, (E, E), 0)
        cols = lax.broadcasted_iota(jnp.int32, (E, E), 1)
        eye = (rows == cols).astype(jnp.float32)

        def fire_in(b, p):
            pltpu.async_copy(
                x_hbm.at[:, pl.ds(b * _BLK, _BLK)], xs[p], sis[p])

        def wait_in(p):
            pltpu.make_async_copy(
                x_hbm.at[:, pl.ds(0, _BLK)], xs[p], sis[p]).wait()

        def step(b, p, first):
            wait_in(p)
            y = xs[p][...].T
            @pl.when(jnp.logical_not(first))
            def _():
                pltpu.make_async_copy(
                    ys[p], out_hbm.at[pl.ds(0, _BLK), :], sos[p]).wait()
            ys[p][:, :E] = y
            ys[p][:, E:] = jnp.zeros((_BLK, E), jnp.float32)
            pltpu.async_copy(
                ys[p], out_hbm.at[pl.ds(b * _BLK, _BLK), :], sos[p])

        fire_in(0, 0)
        fire_in(1, 1)

        def pair(i):
            b0 = 2 * i
            step(b0, 0, i == 0)
            @pl.when(b0 + 2 < nfull)
            def _():
                fire_in(b0 + 2, 0)
            step(b0 + 1, 1, i == 0)
            @pl.when(b0 + 3 < nfull)
            def _():
                fire_in(b0 + 3, 1)

        pl.loop(0, nfull // 2)(pair)

        base = nfull * _BLK
        pltpu.async_copy(
            x_hbm.at[:, pl.ds(base, tail)], xs[1].at[:, pl.ds(0, tail)], si1)
        pltpu.make_async_copy(
            x_hbm.at[:, pl.ds(0, tail)], xs[1].at[:, pl.ds(0, tail)],
            si1).wait()
        yt = xs[1][:, pl.ds(0, tail)].T
        pltpu.make_async_copy(
            ys[1], out_hbm.at[pl.ds(0, _BLK), :], so1).wait()
        ys[1][:tail, :E] = yt
        ys[1][:tail, E:] = jnp.zeros((tail, E), jnp.float32)
        pltpu.async_copy(
            ys[1].at[pl.ds(0, tail), :], out_hbm.at[pl.ds(base, tail), :],
            so1)
        pltpu.sync_copy(patch_hbm, out_hbm.at[pl.ds(base + tail, rag), :])
        pltpu.make_async_copy(
            ys[0], out_hbm.at[pl.ds(0, _BLK), :], so0).wait()
        pltpu.make_async_copy(
            ys[1].at[pl.ds(0, tail), :], out_hbm.at[pl.ds(0, tail), :],
            so1).wait()

    patch = jnp.pad(
        jax.lax.dot_general(
            embT[:, V - rag:], jnp.eye(E, dtype=jnp.float32),
            (((0,), (0,)), ((), ())),
            precision=jax.lax.Precision.HIGHEST),
        ((0, 0), (0, E)))
    out = pl.pallas_call(
        body,
        in_specs=[pl.BlockSpec(memory_space=pl.ANY),
                  pl.BlockSpec(memory_space=pl.ANY)],
        out_specs=pl.BlockSpec(memory_space=pl.ANY),
        out_shape=jax.ShapeDtypeStruct((V, 2 * E), jnp.float32),
        scratch_shapes=[
            pltpu.VMEM((E, _BLK), jnp.float32),
            pltpu.VMEM((E, _BLK), jnp.float32),
            pltpu.VMEM((_BLK, 2 * E), jnp.float32),
            pltpu.VMEM((_BLK, 2 * E), jnp.float32),
            pltpu.SemaphoreType.DMA,
            pltpu.SemaphoreType.DMA,
            pltpu.SemaphoreType.DMA,
            pltpu.SemaphoreType.DMA,
        ],
    )(embT, patch)
    return out.reshape(2 * V, E)


def kernel(input, emb_weight):
    emb2 = _relayout(emb_weight.T)
    return _bow_encode(input.astype(jnp.int32) * 2, emb2)


# speedup vs baseline: 2.6643x; 1.1444x over previous
import functools

import jax
import jax.numpy as jnp
from jax import lax
from jax.experimental import pallas as pl
from jax.experimental.pallas import tpu as pltpu
from jax.experimental.pallas import tpu_sc as plsc

NUM_CORES = 2
NUM_SUBCORES = 16
NUM_WORKERS = NUM_CORES * NUM_SUBCORES
LANES = 16


@jax.jit
def _bow_encode(input_ids, emb_weight):
    B, S = input_ids.shape
    V, E = emb_weight.shape
    b_per_w = B // NUM_WORKERS
    n_chunks = 5
    chunk = S // n_chunks
    n_col = E // LANES

    mesh = plsc.VectorSubcoreMesh(
        core_axis_name="c", subcore_axis_name="s",
        num_cores=NUM_CORES, num_subcores=NUM_SUBCORES,
    )

    @functools.partial(
        pl.kernel,
        out_type=jax.ShapeDtypeStruct((B, E), jnp.float32),
        mesh=mesh,
        scratch_types=[
            pltpu.VMEM((b_per_w, S), jnp.int32),
            pltpu.VMEM((S, E), jnp.float32),
            pltpu.VMEM((S, E), jnp.float32),
            pltpu.VMEM((b_per_w, E), jnp.float32),
            pltpu.SemaphoreType.DMA,
            pltpu.SemaphoreType.DMA,
        ],
        compiler_params=pltpu.CompilerParams(use_tc_tiling_on_sc=False),
    )
    def bow_kernel(idx_hbm, table_hbm, out_hbm, idx_v, buf0, buf1, out_v,
                   sem0, sem1):
        wid = lax.axis_index("s") * NUM_CORES + lax.axis_index("c")
        base = wid * b_per_w
        pltpu.sync_copy(idx_hbm.at[pl.ds(base, b_per_w), :], idx_v)

        bufs = (buf0, buf1)
        sems = (sem0, sem1)

        def fire(b, p):
            for j in range(n_chunks):
                pltpu.async_copy(
                    table_hbm.at[idx_v.at[b, pl.ds(j * chunk, chunk)]],
                    bufs[p].at[pl.ds(j * chunk, chunk), :],
                    sems[p],
                )

        def drain(p):
            pltpu.make_async_copy(
                table_hbm.at[pl.ds(0, S), :], bufs[p], sems[p]).wait()

        def reduce(p, b):
            buf = bufs[p]

            def body(s, accs):
                return tuple(
                    jnp.maximum(a, buf[s, pl.ds(j * LANES, LANES)])
                    for j, a in enumerate(accs)
                )

            accs = tuple(buf[0, pl.ds(j * LANES, LANES)] for j in range(n_col))
            accs = lax.fori_loop(1, S, body, accs, unroll=4)
            for j in range(n_col):
                out_v[b, pl.ds(j * LANES, LANES)] = accs[j]

        fire(0, 0)

        def pair_body(i):
            b0 = 2 * i
            fire(b0 + 1, 1)
            drain(0)
            reduce(0, b0)
            fire(b0 + 2, 0)
            drain(1)
            reduce(1, b0 + 1)

        pl.loop(0, b_per_w // 2 - 1)(pair_body)

        b0 = b_per_w - 2
        fire(b0 + 1, 1)
        drain(0)
        reduce(0, b0)
        drain(1)
        reduce(1, b0 + 1)

        pltpu.sync_copy(out_v, out_hbm.at[pl.ds(base, b_per_w), :])

    return bow_kernel(input_ids, emb_weight)


_BLK = 8192


@jax.jit
def _relayout(embT):
    E, V = embT.shape
    nfull = V // _BLK
    tail = (V - nfull * _BLK) // 128 * 128
    rag = V - nfull * _BLK - tail

    def body(x_hbm, patch_hbm, out_hbm, x0, x1, y0, y1, si0, si1, so0, so1):
        xs, ys = (x0, x1), (y0, y1)
        sis, sos = (si0, si1), (so0, so1)

        rows = lax.broadcasted_iota(jnp.int32, (E, E), 0)
        cols = lax.broadcasted_iota(jnp.int32, (E, E), 1)
        eye = (rows == cols).astype(jnp.float32)

        def fire_in(b, p):
            pltpu.async_copy(
                x_hbm.at[:, pl.ds(b * _BLK, _BLK)], xs[p], sis[p])

        def wait_in(p):
            pltpu.make_async_copy(
                x_hbm.at[:, pl.ds(0, _BLK)], xs[p], sis[p]).wait()

        def step(b, p, first):
            wait_in(p)
            y = xs[p][...].T
            @pl.when(jnp.logical_not(first))
            def _():
                pltpu.make_async_copy(
                    ys[p], out_hbm.at[pl.ds(0, _BLK), :], sos[p]).wait()
            ys[p][:, :E] = y
            ys[p][:, E:] = jnp.zeros((_BLK, E), jnp.float32)
            pltpu.async_copy(
                ys[p], out_hbm.at[pl.ds(b * _BLK, _BLK), :], sos[p])

        fire_in(0, 0)
        fire_in(1, 1)

        def pair(i):
            b0 = 2 * i
            step(b0, 0, i == 0)
            @pl.when(b0 + 2 < nfull)
            def _():
                fire_in(b0 + 2, 0)
            step(b0 + 1, 1, i == 0)
            @pl.when(b0 + 3 < nfull)
            def _():
                fire_in(b0 + 3, 1)

        pl.loop(0, nfull // 2)(pair)

        base = nfull * _BLK
        pltpu.async_copy(
            x_hbm.at[:, pl.ds(base, tail)], xs[1].at[:, pl.ds(0, tail)], si1)
        pltpu.make_async_copy(
            x_hbm.at[:, pl.ds(0, tail)], xs[1].at[:, pl.ds(0, tail)],
            si1).wait()
        yt = xs[1][:, pl.ds(0, tail)].T
        pltpu.make_async_copy(
            ys[1], out_hbm.at[pl.ds(0, _BLK), :], so1).wait()
        ys[1][:tail, :E] = yt
        ys[1][:tail, E:] = jnp.zeros((tail, E), jnp.float32)
        pltpu.async_copy(
            ys[1].at[pl.ds(0, tail), :], out_hbm.at[pl.ds(base, tail), :],
            so1)
        pltpu.sync_copy(patch_hbm, out_hbm.at[pl.ds(base + tail, rag), :])
        pltpu.make_async_copy(
            ys[0], out_hbm.at[pl.ds(0, _BLK), :], so0).wait()
        pltpu.make_async_copy(
            ys[1].at[pl.ds(0, tail), :], out_hbm.at[pl.ds(0, tail), :],
            so1).wait()

    patch = jnp.pad(
        jax.lax.dot_general(
            embT[:, V - rag:], jnp.eye(E, dtype=jnp.float32),
            (((0,), (0,)), ((), ())),
            precision=jax.lax.Precision.HIGHEST),
        ((0, 0), (0, E)))
    out = pl.pallas_call(
        body,
        in_specs=[pl.BlockSpec(memory_space=pl.ANY),
                  pl.BlockSpec(memory_space=pl.ANY)],
        out_specs=pl.BlockSpec(memory_space=pl.ANY),
        out_shape=jax.ShapeDtypeStruct((V, 2 * E), jnp.float32),
        scratch_shapes=[
            pltpu.VMEM((E, _BLK), jnp.float32),
            pltpu.VMEM((E, _BLK), jnp.float32),
            pltpu.VMEM((_BLK, 2 * E), jnp.float32),
            pltpu.VMEM((_BLK, 2 * E), jnp.float32),
            pltpu.SemaphoreType.DMA,
            pltpu.SemaphoreType.DMA,
            pltpu.SemaphoreType.DMA,
            pltpu.SemaphoreType.DMA,
        ],
    )(embT, patch)
    return out.reshape(2 * V, E)


def kernel(input, emb_weight):
    emb2 = _relayout(emb_weight.T)
    return _bow_encode(input.astype(jnp.int32) * 2, emb2)
